# baseline (speedup 1.0000x reference)
import jax
import jax.numpy as jnp

_H = 2


def _gat_layer(x, edge_index, W, a_src, a_dst, b, out_ch):
    n = x.shape[0]
    h = (x @ W).reshape(n, _H, out_ch)
    alpha_src = (h * a_src[None, :, :]).sum(-1)
    alpha_dst = (h * a_dst[None, :, :]).sum(-1)
    loop = jnp.arange(n, dtype=edge_index.dtype)
    src = jnp.concatenate([edge_index[0], loop])
    dst = jnp.concatenate([edge_index[1], loop])
    alpha = jax.nn.leaky_relu(alpha_src[src] + alpha_dst[dst], 0.2)
    amax = jax.ops.segment_max(alpha, dst, num_segments=n)
    ex = jnp.exp(alpha - amax[dst])
    den = jax.ops.segment_sum(ex, dst, num_segments=n)
    attn = ex / (den[dst] + 1e-16)
    out = jax.ops.segment_sum(h[src] * attn[..., None], dst, num_segments=n)
    out = out.reshape(n, _H * out_ch) + b
    return out


def kernel(x, undirected_edges_middle, directed_edges_middle,
           W1, a_src1, a_dst1, b1, W2, a_src2, a_dst2, b2):
    h1 = _gat_layer(x, undirected_edges_middle, W1, a_src1, a_dst1, b1, 128)
    h2 = _gat_layer(h1, directed_edges_middle, W2, a_src2, a_dst2, b2, 256)
    return h2


# speedup vs baseline: 7.2051x; 7.2051x over previous
import functools

import jax
import jax.numpy as jnp
from jax import lax
from jax.experimental import pallas as pl
from jax.experimental.pallas import tpu as pltpu
from jax.experimental.pallas import tpu_sc as plsc

_N = 10000
_E = 320000
_H = 2
_NSC = 2
_NT = 16
_SEG = 2000
_EPT = _E // _NT
_NSEG = _EPT // _SEG


def _mm_split_body(x_ref, w_ref, h_ref, al_ref):
    res = jnp.dot(x_ref[...], w_ref[...], preferred_element_type=jnp.float32)
    c = h_ref.shape[1]
    h_ref[...] = res[:, :c]
    al_ref[...] = res[:, c:c + 8]


def _mm_split(x, w, c, block_rows=1000):
    n, k = x.shape
    return pl.pallas_call(
        _mm_split_body,
        grid=(n // block_rows,),
        in_specs=[pl.BlockSpec((block_rows, k), lambda i: (i, 0)),
                  pl.BlockSpec((k, w.shape[1]), lambda i: (0, 0))],
        out_specs=[pl.BlockSpec((block_rows, c), lambda i: (i, 0)),
                   pl.BlockSpec((block_rows, 8), lambda i: (i, 0))],
        out_shape=[jax.ShapeDtypeStruct((n, c), jnp.float32),
                   jax.ShapeDtypeStruct((n, 8), jnp.float32)],
    )(x, w)


def _normalize(acc_ref, h_ref, al_ref, den_ref, b_ref):
    ch = acc_ref.shape[1] // _H
    den = den_ref[...]
    al = al_ref[...]
    a = al[:, 0:_H] + al[:, _H:2 * _H]
    exl = jnp.exp(jnp.where(a >= 0, a, 0.2 * a))
    cols = []
    for k in range(_H):
        num = acc_ref[:, k * ch:(k + 1) * ch] + exl[:, k:k + 1] * h_ref[:, k * ch:(k + 1) * ch]
        cols.append(num / (den[:, k:k + 1] + exl[:, k:k + 1]))
    return jnp.concatenate(cols, axis=1) + b_ref[...]


def _norm_mm_body(acc_ref, h_ref, al_ref, den_ref, b_ref, w_ref,
                  h2_ref, al2_ref):
    x2 = _normalize(acc_ref, h_ref, al_ref, den_ref, b_ref)
    res = jnp.dot(x2, w_ref[...], preferred_element_type=jnp.float32)
    c2 = h2_ref.shape[1]
    h2_ref[...] = res[:, :c2]
    al2_ref[...] = res[:, c2:c2 + 8]


def _norm_mm(acc, h, al, den2, b, w, c2, block_rows=1000):
    c = acc.shape[1]
    nb = _N // block_rows
    return pl.pallas_call(
        _norm_mm_body,
        grid=(nb,),
        in_specs=[
            pl.BlockSpec((block_rows, c), lambda i: (i, 0)),
            pl.BlockSpec((block_rows, c), lambda i: (i, 0)),
            pl.BlockSpec((block_rows, 8), lambda i: (i, 0)),
            pl.BlockSpec((block_rows, _H), lambda i: (i, 0)),
            pl.BlockSpec((1, c), lambda i: (0, 0)),
            pl.BlockSpec((c, w.shape[1]), lambda i: (0, 0)),
        ],
        out_specs=[pl.BlockSpec((block_rows, c2), lambda i: (i, 0)),
                   pl.BlockSpec((block_rows, 8), lambda i: (i, 0))],
        out_shape=[jax.ShapeDtypeStruct((_N, c2), jnp.float32),
                   jax.ShapeDtypeStruct((_N, 8), jnp.float32)],
    )(acc, h, al, den2, b, w)


def _norm_final_body(acc_ref, h_ref, al_ref, den_ref, b_ref, o_ref):
    o_ref[...] = _normalize(acc_ref, h_ref, al_ref, den_ref, b_ref)


def _norm_final(acc, h, al, den2, b, block_rows=1000):
    c = acc.shape[1]
    nb = _N // block_rows
    return pl.pallas_call(
        _norm_final_body,
        grid=(nb,),
        in_specs=[
            pl.BlockSpec((block_rows, c), lambda i: (i, 0)),
            pl.BlockSpec((block_rows, c), lambda i: (i, 0)),
            pl.BlockSpec((block_rows, 8), lambda i: (i, 0)),
            pl.BlockSpec((block_rows, _H), lambda i: (i, 0)),
            pl.BlockSpec((1, c), lambda i: (0, 0)),
        ],
        out_specs=pl.BlockSpec((block_rows, c), lambda i: (i, 0)),
        out_shape=jax.ShapeDtypeStruct((_N, c), jnp.float32),
    )(acc, h, al, den2, b)


def _make_edge_kernel(c, rt, np_):
    nw = _NSC * _NT
    tot = np_ * nw * rt
    mesh = plsc.VectorSubcoreMesh(core_axis_name="c", subcore_axis_name="s")

    def body(h_hbm, src_hbm, dst_hbm, asrc_hbm, adst_hbm,
             acc_out, den_out,
             asrc_t, adst_t, seg_s, seg_d, f_src, f_dst,
             exb0, exb1, den_t, stage, gidx_v, acc_t, sem):
        wid = lax.axis_index("c") * _NT + lax.axis_index("s")

        pltpu.sync_copy(asrc_hbm, asrc_t)
        pltpu.sync_copy(adst_hbm, adst_t)

        zero16f = jnp.zeros((16,), jnp.float32)
        iota16 = lax.iota(jnp.int32, 16)
        lane0 = iota16 == 0
        cols = [iota16 + 16 * j for j in range(c // 16)]

        for p in range(np_):
            lo = (p * nw + wid) * rt

            def zacc(r, carry):
                for j in range(c // 16):
                    acc_t[r, pl.ds(16 * j, 16)] = zero16f
                return carry
            lax.fori_loop(0, rt + 8, zacc, 0)

            def zden(i, carry):
                den_t[pl.ds(i * 16, 16)] = zero16f
                return carry
            lax.fori_loop(0, (rt + 8) * _H // 16, zden, 0)

            def seg_body(g, carry):
                pltpu.sync_copy(src_hbm.at[pl.ds(g * _SEG, _SEG)], seg_s)
                pltpu.sync_copy(dst_hbm.at[pl.ds(g * _SEG, _SEG)], seg_d)

                def vreg_body(j, cnt):
                    src16 = seg_s[pl.ds(j * 16, 16)]
                    dst16 = seg_d[pl.ds(j * 16, 16)]
                    dloc = dst16 - lo
                    m = dloc.astype(jnp.uint32) < jnp.uint32(rt)
                    plsc.store_compressed(f_src.at[pl.ds(cnt, 16)], src16,
                                          mask=m)
                    plsc.store_compressed(f_dst.at[pl.ds(cnt, 16)], dloc,
                                          mask=m)
                    return cnt + jnp.sum(m.astype(jnp.int32))

                cnt = lax.fori_loop(0, _SEG // 16, vreg_body, 0)

                for k in range(2):
                    f_src[pl.ds(cnt + k * 16, 16)] = jnp.zeros((16,), jnp.int32)
                    f_dst[pl.ds(cnt + k * 16, 16)] = jnp.full((16,), rt, jnp.int32)
                nb = (cnt + 31) // 32

                def batch_body(b_, carry):
                    for k in range(2):
                        off = b_ * 32 + k * 16
                        s16 = f_src[pl.ds(off, 16)]
                        dl16 = f_dst[pl.ds(off, 16)]
                        gidx_v[pl.ds(k * 16, 16)] = s16
                        dn2 = jnp.minimum(dl16 + lo, _N - 1) * 2
                        s2 = s16 * 2
                        a0 = plsc.load_gather(asrc_t, [s2]) + \
                            plsc.load_gather(adst_t, [dn2])
                        a1 = plsc.load_gather(asrc_t, [s2 + 1]) + \
                            plsc.load_gather(adst_t, [dn2 + 1])
                        exb0[pl.ds(k * 16, 16)] = jnp.exp(
                            jnp.where(a0 >= 0, a0, 0.2 * a0))
                        exb1[pl.ds(k * 16, 16)] = jnp.exp(
                            jnp.where(a1 >= 0, a1, 0.2 * a1))
                    pltpu.async_copy(h_hbm.at[gidx_v], stage, sem).wait()

                    def row_body(r, carry2):
                        rsplat = jnp.broadcast_to(r, (16,))
                        dl = plsc.load_gather(f_dst,
                                              [jnp.broadcast_to(b_ * 32 + r, (16,))])
                        e0 = plsc.load_gather(exb0, [rsplat])
                        e1 = plsc.load_gather(exb1, [rsplat])
                        dl2 = dl * 2
                        plsc.addupdate_scatter(den_t, [dl2], e0, mask=lane0)
                        plsc.addupdate_scatter(den_t, [dl2 + 1], e1, mask=lane0)
                        for j in range(c // 32):
                            v = stage[r, pl.ds(16 * j, 16)] * e0
                            plsc.addupdate_scatter(acc_t, [dl, cols[j]], v)
                        for j in range(c // 32, c // 16):
                            v = stage[r, pl.ds(16 * j, 16)] * e1
                            plsc.addupdate_scatter(acc_t, [dl, cols[j]], v)
                        return carry2
                    lax.fori_loop(0, 32, row_body, 0)
                    return carry
                lax.fori_loop(0, nb, batch_body, 0)
                return carry
            lax.fori_loop(0, _E // _SEG, seg_body, 0)

            pltpu.sync_copy(acc_t.at[pl.ds(0, rt)], acc_out.at[pl.ds(lo, rt)])
            pltpu.sync_copy(den_t.at[pl.ds(0, rt * _H)],
                            den_out.at[pl.ds(lo * _H, rt * _H)])

    out_type = [
        jax.ShapeDtypeStruct((tot, c), jnp.float32),
        jax.ShapeDtypeStruct((tot * _H,), jnp.float32),
    ]
    scratch = [
        pltpu.VMEM((_N * _H,), jnp.float32),
        pltpu.VMEM((_N * _H,), jnp.float32),
        pltpu.VMEM((_SEG,), jnp.int32),
        pltpu.VMEM((_SEG,), jnp.int32),
        pltpu.VMEM((_SEG + 32,), jnp.int32),
        pltpu.VMEM((_SEG + 32,), jnp.int32),
        pltpu.VMEM((32,), jnp.float32),
        pltpu.VMEM((32,), jnp.float32),
        pltpu.VMEM(((rt + 8) * _H,), jnp.float32),
        pltpu.VMEM((32, c), jnp.float32),
        pltpu.VMEM((32,), jnp.int32),
        pltpu.VMEM((rt + 8, c), jnp.float32),
        pltpu.SemaphoreType.DMA,
    ]
    return functools.partial(
        pl.kernel, mesh=mesh, out_type=out_type, scratch_types=scratch,
        compiler_params=pltpu.CompilerParams(needs_layout_passes=False),
    )(body)


def _big_weight(W, a_src, a_dst, out_ch):
    wa_src = (W.reshape(-1, _H, out_ch) * a_src[None]).sum(-1)
    wa_dst = (W.reshape(-1, _H, out_ch) * a_dst[None]).sum(-1)
    pad = jnp.zeros((W.shape[0], 8 - 2 * _H), jnp.float32)
    return jnp.concatenate([W, wa_src, wa_dst, pad], axis=1)


def kernel(x, undirected_edges_middle, directed_edges_middle,
           W1, a_src1, a_dst1, b1, W2, a_src2, a_dst2, b2):
    big1 = _big_weight(W1, a_src1, a_dst1, 128)
    h1, al1 = _mm_split(x, big1, _H * 128)

    edge1 = _make_edge_kernel(_H * 128, 240, 2)
    acc1, den1 = edge1(h1,
                       undirected_edges_middle[0], undirected_edges_middle[1],
                       al1[:, 0:_H].reshape(_N * _H),
                       al1[:, _H:2 * _H].reshape(_N * _H))
    den1 = den1.reshape(-1, _H)

    big2 = _big_weight(W2, a_src2, a_dst2, 256)
    h2, al2 = _norm_mm(acc1, h1, al1, den1, b1.reshape(1, -1), big2, _H * 256)

    edge2 = _make_edge_kernel(_H * 256, 112, 3)
    acc2, den2 = edge2(h2,
                       directed_edges_middle[0], directed_edges_middle[1],
                       al2[:, 0:_H].reshape(_N * _H),
                       al2[:, _H:2 * _H].reshape(_N * _H))
    den2 = den2.reshape(-1, _H)

    return _norm_final(acc2, h2, al2, den2, b2.reshape(1, -1))
